# baseline (speedup 1.0000x reference)
import jax
import jax.numpy as jnp
from jax import lax
from jax.experimental import pallas as pl
from jax.experimental.pallas import tpu as pltpu
from jax.experimental.pallas import tpu_sc as plsc

B = 4096
N = 20
D = 64
V = 1000000
NC = 2
NS = 16
L = 16
NW = NC * NS

GFULL = V // 128
GPW = GFULL // NW
GEXTRA = GFULL - GPW * NW
VTAIL = V - GFULL * 128
GB = 4
RING = 4
GSTEPS = GPW // GB

BPW = B // NW
CB = 32
NCHUNK = BPW // CB
ROWS = CB * N
NGRP = ROWS // 128
NHALF = N // 2


def _worker_id():
    return lax.axis_index("s") * NC + lax.axis_index("c")


def _transpose_group(stage, blk, n_e, e_base):
    lane = lax.iota(jnp.int32, L)
    b_idx = lane % 8
    a_idx = [lane // 8 + 2 * k for k in range(4)]

    @plsc.parallel_loop(0, n_e, step=1, unroll=4)
    def _ebody(e):
        e_idx = jnp.full((L,), e_base, jnp.int32) + e
        vals = [plsc.load_gather(stage, [a_idx[k], b_idx, e_idx])
                for k in range(4)]
        for k in range(4):
            blk[pl.ds(e * D + k * L, L)] = vals[k]


def _relayout_body(tab3_hbm, tail_hbm, rows_hbm, stages, blks, tail_v,
                   sems_i, sems_o):
    wid = _worker_id()
    cb = wid * GPW

    def start_in(c0, s, width):
        for a in range(8):
            pltpu.make_async_copy(
                tab3_hbm.at[a, :, pl.ds(c0 * 128, width * 128)],
                stages[s].at[a, :, pl.ds(0, width * 128)],
                sems_i[s],
            ).start()

    def wait_in(s, width):
        for a in range(8):
            pltpu.make_async_copy(
                tab3_hbm.at[a, :, pl.ds(0, width * 128)],
                stages[s].at[a, :, pl.ds(0, width * 128)],
                sems_i[s],
            ).wait()

    def start_out(c, s):
        pltpu.make_async_copy(
            blks[s], rows_hbm.at[pl.ds(c * 8192, 8192)], sems_o[s]
        ).start()

    def wait_out(s):
        pltpu.make_async_copy(
            blks[s], rows_hbm.at[pl.ds(0, 8192)], sems_o[s]
        ).wait()

    start_in(cb, 0, GB)

    def tbody(t2, _):
        for ss in range(2):
            t = t2 * 2 + ss
            c0 = cb + t * GB
            wait_in(ss, GB)
            start_in(c0 + GB, 1 - ss, GB)
            for g in range(GB):
                @pl.when(t > 0)
                def _():
                    wait_out(g)

                _transpose_group(stages[ss], blks[g], 128, g * 128)
                start_out(c0 + g, g)
        return 0

    lax.fori_loop(0, GSTEPS // 2, tbody, 0)

    c0 = cb + (GSTEPS - 1) * GB
    wait_in(0, GB)
    for g in range(GB):
        wait_out(g)
        _transpose_group(stages[0], blks[g], 128, g * 128)
        start_out(c0 + g, g)
    for g in range(GB):
        wait_out(g)

    @pl.when(wid >= NW - GEXTRA)
    def _():
        c = GPW * NW + (wid - (NW - GEXTRA))
        start_in(c, 0, 1)
        wait_in(0, 1)
        _transpose_group(stages[0], blks[0], 128, 0)
        start_out(c, 0)
        wait_out(0)

    @pl.when(wid == NW - GEXTRA - 1)
    def _():
        pltpu.sync_copy(tail_hbm, tail_v)
        pltpu.sync_copy(tail_v, rows_hbm.at[pl.ds(GFULL * 8192, VTAIL * D)])


def _score_body(ctx_hbm, ids_hbm, rows2d_hbm, out_hbm,
                ids_v, ctx_v, rows_v, scores_v, out_v, sem_g, sems_o):
    wid = _worker_id()
    b0 = wid * BPW

    pltpu.sync_copy(ctx_hbm.at[pl.ds(b0, BPW), :], ctx_v)
    pltpu.sync_copy(ids_hbm.at[pl.ds(b0 * N, BPW * N)], ids_v)

    def fire_gathers(chunk, slot):
        for g in range(NGRP):
            pltpu.make_async_copy(
                rows2d_hbm.at[ids_v.at[pl.ds(chunk * ROWS + g * 128, 128)]],
                rows_v[slot].at[pl.ds(g * 128, 128), :],
                sem_g[slot],
            ).start()

    def wait_gathers(slot):
        for g in range(NGRP):
            pltpu.make_async_copy(
                rows2d_hbm.at[ids_v.at[pl.ds(g * 128, 128)]],
                rows_v[slot].at[pl.ds(g * 128, 128), :],
                sem_g[slot],
            ).wait()

    fire_gathers(0, 0)
    fire_gathers(1, 1)

    lane = lax.iota(jnp.int32, L)

    for chunk in range(NCHUNK):
        slot = chunk % 2
        wait_gathers(slot)

        for bb in range(CB // L):
            bcol = lane + (chunk * CB + bb * L)
            rbase = (lane + bb * L) * N

            for h in range(N // NHALF):
                def dbody(d, accs, _h=h, _bcol=bcol, _rbase=rbase):
                    dcol = jnp.full((L,), 0, jnp.int32) + d
                    cvec = plsc.load_gather(ctx_v, [_bcol, dcol])
                    return tuple(
                        accs[i]
                        + plsc.load_gather(
                            rows_v[slot], [_rbase + (_h * NHALF + i), dcol])
                        * cvec
                        for i in range(NHALF)
                    )

                accs = lax.fori_loop(
                    0, D, dbody,
                    tuple(jnp.zeros((L,), jnp.float32) for _ in range(NHALF)))
                for i in range(NHALF):
                    scores_v[h * NHALF + i, :] = accs[i]

            m = scores_v[0, :]
            for n in range(1, N):
                m = jnp.maximum(m, scores_v[n, :])
            tot = jnp.zeros((L,), jnp.float32)
            es = []
            for n in range(N):
                e = jnp.exp(scores_v[n, :] - m)
                es.append(e)
                tot = tot + e
            r = 1.0 / tot
            blocal = lane + bb * L
            for n in range(N):
                ncol = jnp.full((L,), n, jnp.int32)
                plsc.store_scatter(out_v[slot], [blocal, ncol], es[n] * r)

        @pl.when(chunk >= 2)
        def _():
            pltpu.make_async_copy(
                out_v[slot], out_hbm.at[pl.ds(0, CB), :], sems_o[slot]
            ).wait()

        pltpu.make_async_copy(
            out_v[slot], out_hbm.at[pl.ds(b0 + chunk * CB, CB), :],
            sems_o[slot],
        ).start()

        if chunk + 2 < NCHUNK:
            fire_gathers(chunk + 2, slot)

    for slot in range(2):
        pltpu.make_async_copy(
            out_v[slot], out_hbm.at[pl.ds(0, CB), :], sems_o[slot]
        ).wait()


def _mesh():
    return plsc.VectorSubcoreMesh(
        core_axis_name="c", subcore_axis_name="s",
        num_cores=NC, num_subcores=NS)


@jax.jit
def _entity_posterior_sc(context_encoded, ids_flat, tab3, tail_flat):
    def relayout_wrap(tab3_hbm, tail_hbm, rows_hbm, s0, s1,
                      b0, b1, b2, b3, tv, si0, si1,
                      so0, so1, so2, so3):
        _relayout_body(tab3_hbm, tail_hbm, rows_hbm, (s0, s1),
                       (b0, b1, b2, b3), tv, (si0, si1),
                       (so0, so1, so2, so3))

    rows_lin = pl.kernel(
        relayout_wrap,
        out_type=jax.ShapeDtypeStruct((V * D,), jnp.float32),
        mesh=_mesh(),
        scratch_types=(
            [pltpu.VMEM((8, 8, GB * 128), jnp.float32) for _ in range(2)]
            + [pltpu.VMEM((8192,), jnp.float32) for _ in range(RING)]
            + [pltpu.VMEM((VTAIL * D,), jnp.float32)]
            + [pltpu.SemaphoreType.DMA for _ in range(2)]
            + [pltpu.SemaphoreType.DMA for _ in range(RING)]
        ),
        compiler_params=pltpu.CompilerParams(needs_layout_passes=False),
        name="entity_table_relayout_sc",
    )(tab3, tail_flat)

    def score_wrap(ctx_hbm, ids_hbm, rows2d_hbm, out_hbm,
                   ids_v, ctx_v, r0, r1, o0, o1, sv, sg0, sg1, so0, so1):
        _score_body(ctx_hbm, ids_hbm, rows2d_hbm, out_hbm,
                    ids_v, ctx_v, (r0, r1), sv, (o0, o1),
                    (sg0, sg1), (so0, so1))

    return pl.kernel(
        score_wrap,
        out_type=jax.ShapeDtypeStruct((B, N), jnp.float32),
        mesh=_mesh(),
        scratch_types=(
            [pltpu.VMEM((BPW * N,), jnp.int32),
             pltpu.VMEM((BPW, D), jnp.float32)]
            + [pltpu.VMEM((ROWS, D), jnp.float32) for _ in range(2)]
            + [pltpu.VMEM((CB, N), jnp.float32) for _ in range(2)]
            + [pltpu.VMEM((N, L), jnp.float32)]
            + [pltpu.SemaphoreType.DMA for _ in range(4)]
        ),
        compiler_params=pltpu.CompilerParams(
            needs_layout_passes=False, use_tc_tiling_on_sc=False),
        name="entity_posterior_sc",
    )(context_encoded, ids_flat, rows_lin.reshape(V, D))


def kernel(context_encoded, entity_ids, entity_embeddings):
    tab3 = entity_embeddings.T.reshape(8, 8, V)
    tail_flat = entity_embeddings[GFULL * 128:, :].reshape(-1)
    ids_flat = entity_ids.reshape(-1)
    return _entity_posterior_sc(context_encoded, ids_flat, tab3, tail_flat)


# speedup vs baseline: 3.4865x; 3.4865x over previous
import jax
import jax.numpy as jnp
from jax import lax
from jax.experimental import pallas as pl
from jax.experimental.pallas import tpu as pltpu
from jax.experimental.pallas import tpu_sc as plsc

B = 4096
N = 20
D = 64
V = 1000000
NC = 2
NS = 16
L = 16
NW = NC * NS

GFULL = V // 128
GPW = GFULL // NW
GEXTRA = GFULL - GPW * NW
VTAIL = V - GFULL * 128
GB = 4
RING = 4
GSTEPS = GPW // GB

BPW = B // NW
CB = 32
NCHUNK = BPW // CB
ROWS = CB * N
NGRP = ROWS // 128
NHALF = N // 2


def _worker_id():
    return lax.axis_index("s") * NC + lax.axis_index("c")


def _perms():
    lane = lax.iota(jnp.int32, L)
    return lane, [(lane + r) % L for r in range(L)]


def _transpose_group(stage, blk, n_e, e_base):
    lane, perms = _perms()
    lane64 = lane * D

    @plsc.parallel_loop(0, (n_e // L) * (D // L), step=1, unroll=1)
    def _ebody(m):
        be = m // (D // L)
        bd = m % (D // L)
        e_vec = lane + (e_base + be * L)
        sbase = be * (L * D) + bd * L
        for r in range(L):
            perm = perms[r]
            row = perm + bd * L
            v = plsc.load_gather(stage, [row, e_vec])
            idx = lane64 + (perm + sbase)
            plsc.store_scatter(blk, [idx], v)


def _relayout_body(tab3_hbm, tail_hbm, rows_hbm, stages, blks, tail_v,
                   sems_i, sems_o):
    wid = _worker_id()
    cb = wid * GPW

    def start_in(c0, s, width):
        for a in range(8):
            pltpu.make_async_copy(
                tab3_hbm.at[a, :, pl.ds(c0 * 128, width * 128)],
                stages[s].at[pl.ds(a * 8, 8), pl.ds(0, width * 128)],
                sems_i[s],
            ).start()

    def wait_in(s, width):
        for a in range(8):
            pltpu.make_async_copy(
                tab3_hbm.at[a, :, pl.ds(0, width * 128)],
                stages[s].at[pl.ds(a * 8, 8), pl.ds(0, width * 128)],
                sems_i[s],
            ).wait()

    def start_out(c, s):
        pltpu.make_async_copy(
            blks[s], rows_hbm.at[pl.ds(c * 8192, 8192)], sems_o[s]
        ).start()

    def wait_out(s):
        pltpu.make_async_copy(
            blks[s], rows_hbm.at[pl.ds(0, 8192)], sems_o[s]
        ).wait()

    start_in(cb, 0, GB)

    def tbody(t2, _):
        for ss in range(2):
            t = t2 * 2 + ss
            c0 = cb + t * GB
            wait_in(ss, GB)
            start_in(c0 + GB, 1 - ss, GB)
            for g in range(GB):
                @pl.when(t > 0)
                def _():
                    wait_out(g)

                _transpose_group(stages[ss], blks[g], 128, g * 128)
                start_out(c0 + g, g)
        return 0

    lax.fori_loop(0, GSTEPS // 2, tbody, 0)

    c0 = cb + (GSTEPS - 1) * GB
    wait_in(0, GB)
    for g in range(GB):
        wait_out(g)
        _transpose_group(stages[0], blks[g], 128, g * 128)
        start_out(c0 + g, g)
    for g in range(GB):
        wait_out(g)

    @pl.when(wid >= NW - GEXTRA)
    def _():
        c = GPW * NW + (wid - (NW - GEXTRA))
        start_in(c, 0, 1)
        wait_in(0, 1)
        _transpose_group(stages[0], blks[0], 128, 0)
        start_out(c, 0)
        wait_out(0)

    @pl.when(wid == NW - GEXTRA - 1)
    def _():
        pltpu.sync_copy(tail_hbm, tail_v)
        pltpu.sync_copy(tail_v, rows_hbm.at[pl.ds(GFULL * 8192, VTAIL * D)])


def _score_body(ctx_hbm, ids_hbm, rows2d_hbm, out_hbm,
                ids_v, ctx_v, rows_v, scores_v, out_v, sem_g, sems_o):
    wid = _worker_id()
    b0 = wid * BPW

    pltpu.sync_copy(ctx_hbm.at[pl.ds(b0, BPW), :], ctx_v)
    pltpu.sync_copy(ids_hbm.at[pl.ds(b0 * N, BPW * N)], ids_v)

    def fire_gathers(chunk, slot):
        for g in range(NGRP):
            pltpu.make_async_copy(
                rows2d_hbm.at[ids_v.at[pl.ds(chunk * ROWS + g * 128, 128)]],
                rows_v[slot].at[pl.ds(g * 128, 128), :],
                sem_g[slot],
            ).start()

    def wait_gathers(slot):
        for g in range(NGRP):
            pltpu.make_async_copy(
                rows2d_hbm.at[ids_v.at[pl.ds(g * 128, 128)]],
                rows_v[slot].at[pl.ds(g * 128, 128), :],
                sem_g[slot],
            ).wait()

    fire_gathers(0, 0)
    fire_gathers(1, 1)

    lane, perms = _perms()

    for chunk in range(NCHUNK):
        slot = chunk % 2
        wait_gathers(slot)

        def bbody(bb, _, _slot=slot, _chunk=chunk):
            bcol = lane + (_chunk * CB + bb * L)
            rbase = (lane + bb * L) * N

            for h in range(N // NHALF):
                rvecs = [rbase + (h * NHALF + i) for i in range(NHALF)]

                def dbody(j, accs, _rv=rvecs, _bcol=bcol):
                    col = ((lane + j) % L) + (j - (j % L))
                    cvec = plsc.load_gather(ctx_v, [_bcol, col])
                    return tuple(
                        accs[i]
                        + plsc.load_gather(rows_v[_slot], [_rv[i], col])
                        * cvec
                        for i in range(NHALF)
                    )

                accs = lax.fori_loop(
                    0, D, dbody,
                    tuple(jnp.zeros((L,), jnp.float32) for _ in range(NHALF)))
                for i in range(NHALF):
                    scores_v[h * NHALF + i, :] = accs[i]

            m = scores_v[0, :]
            for n in range(1, N):
                m = jnp.maximum(m, scores_v[n, :])
            tot = jnp.zeros((L,), jnp.float32)
            for n in range(N):
                e = jnp.exp(scores_v[n, :] - m)
                scores_v[n, :] = e
                tot = tot + e
            r = 1.0 / tot
            blocal = lane + bb * L
            for n in range(N):
                ncol = jnp.full((L,), n, jnp.int32)
                plsc.store_scatter(out_v[_slot], [blocal, ncol],
                                   scores_v[n, :] * r)
            return 0

        lax.fori_loop(0, CB // L, bbody, 0)

        @pl.when(chunk >= 2)
        def _():
            pltpu.make_async_copy(
                out_v[slot], out_hbm.at[pl.ds(0, CB), :], sems_o[slot]
            ).wait()

        pltpu.make_async_copy(
            out_v[slot], out_hbm.at[pl.ds(b0 + chunk * CB, CB), :],
            sems_o[slot],
        ).start()

        if chunk + 2 < NCHUNK:
            fire_gathers(chunk + 2, slot)

    for slot in range(2):
        pltpu.make_async_copy(
            out_v[slot], out_hbm.at[pl.ds(0, CB), :], sems_o[slot]
        ).wait()


def _mesh():
    return plsc.VectorSubcoreMesh(
        core_axis_name="c", subcore_axis_name="s",
        num_cores=NC, num_subcores=NS)


@jax.jit
def _entity_posterior_sc(context_encoded, ids_flat, tab3, tail_flat):
    def relayout_wrap(tab3_hbm, tail_hbm, rows_hbm, s0, s1,
                      b0, b1, b2, b3, tv, si0, si1,
                      so0, so1, so2, so3):
        _relayout_body(tab3_hbm, tail_hbm, rows_hbm, (s0, s1),
                       (b0, b1, b2, b3), tv, (si0, si1),
                       (so0, so1, so2, so3))

    rows_lin = pl.kernel(
        relayout_wrap,
        out_type=jax.ShapeDtypeStruct((V * D,), jnp.float32),
        mesh=_mesh(),
        scratch_types=(
            [pltpu.VMEM((D, GB * 128), jnp.float32) for _ in range(2)]
            + [pltpu.VMEM((8192,), jnp.float32) for _ in range(RING)]
            + [pltpu.VMEM((VTAIL * D,), jnp.float32)]
            + [pltpu.SemaphoreType.DMA for _ in range(2)]
            + [pltpu.SemaphoreType.DMA for _ in range(RING)]
        ),
        compiler_params=pltpu.CompilerParams(needs_layout_passes=False),
        name="entity_table_relayout_sc",
    )(tab3, tail_flat)

    def score_wrap(ctx_hbm, ids_hbm, rows2d_hbm, out_hbm,
                   ids_v, ctx_v, r0, r1, o0, o1, sv, sg0, sg1, so0, so1):
        _score_body(ctx_hbm, ids_hbm, rows2d_hbm, out_hbm,
                    ids_v, ctx_v, (r0, r1), sv, (o0, o1),
                    (sg0, sg1), (so0, so1))

    return pl.kernel(
        score_wrap,
        out_type=jax.ShapeDtypeStruct((B, N), jnp.float32),
        mesh=_mesh(),
        scratch_types=(
            [pltpu.VMEM((BPW * N,), jnp.int32),
             pltpu.VMEM((BPW, D), jnp.float32)]
            + [pltpu.VMEM((ROWS, D), jnp.float32) for _ in range(2)]
            + [pltpu.VMEM((CB, N), jnp.float32) for _ in range(2)]
            + [pltpu.VMEM((N, L), jnp.float32)]
            + [pltpu.SemaphoreType.DMA for _ in range(4)]
        ),
        compiler_params=pltpu.CompilerParams(
            needs_layout_passes=False, use_tc_tiling_on_sc=False),
        name="entity_posterior_sc",
    )(context_encoded, ids_flat, rows_lin.reshape(V, D))


def kernel(context_encoded, entity_ids, entity_embeddings):
    tab3 = entity_embeddings.T.reshape(8, 8, V)
    tail_flat = entity_embeddings[GFULL * 128:, :].reshape(-1)
    ids_flat = entity_ids.reshape(-1)
    return _entity_posterior_sc(context_encoded, ids_flat, tab3, tail_flat)


# speedup vs baseline: 3.8234x; 1.0966x over previous
import jax
import jax.numpy as jnp
from jax import lax
from jax.experimental import pallas as pl
from jax.experimental.pallas import tpu as pltpu
from jax.experimental.pallas import tpu_sc as plsc

B = 4096
N = 20
D = 64
V = 1000000
NC = 2
NS = 16
L = 16
NW = NC * NS

GFULL = V // 128
GPW = GFULL // NW
GEXTRA = GFULL - GPW * NW
VTAIL = V - GFULL * 128
GB = 4
RING = 4
GSTEPS = GPW // GB

BPW = B // NW
CB = 32
NCHUNK = BPW // CB
ROWS = CB * N
NGRP = ROWS // 128
NHALF = N // 2


def _worker_id():
    return lax.axis_index("s") * NC + lax.axis_index("c")


def _perms():
    lane = lax.iota(jnp.int32, L)
    return lane, [(lane + r) % L for r in range(L)]


def _transpose_group(stage, blk, n_e, e_base):
    lane, perms = _perms()
    lane64 = lane * D

    lane32 = lane * (D // 2)

    @plsc.parallel_loop(0, (n_e // L) * 2, step=1, unroll=1)
    def _ebody(m):
        be = m // 2
        qb = m % 2
        e_vec = lane + (e_base + be * L)
        sbase = be * (L * (D // 2)) + qb * L
        for r in range(L):
            perm = perms[r]
            rowe = perm * 2 + qb * 32
            v1 = plsc.load_gather(stage, [rowe, e_vec])
            v2 = plsc.load_gather(stage, [rowe + 1, e_vec])
            w = plsc.bitcast(
                plsc.pack(v1, v2, format=plsc.PackFormat.INTERLEAVED),
                jnp.int32)
            idx = lane32 + (perm + sbase)
            plsc.store_scatter(blk, [idx], w)


def _relayout_body(tab3_hbm, tail_hbm, rows_hbm, stages, blks, tail_v,
                   sems_i, sems_o):
    wid = _worker_id()
    cb = wid * GPW

    def start_in(c0, s, width):
        for a in range(8):
            pltpu.make_async_copy(
                tab3_hbm.at[a, :, pl.ds(c0 * 128, width * 128)],
                stages[s].at[pl.ds(a * 8, 8), pl.ds(0, width * 128)],
                sems_i[s],
            ).start()

    def wait_in(s, width):
        for a in range(8):
            pltpu.make_async_copy(
                tab3_hbm.at[a, :, pl.ds(0, width * 128)],
                stages[s].at[pl.ds(a * 8, 8), pl.ds(0, width * 128)],
                sems_i[s],
            ).wait()

    def start_out(c, s):
        pltpu.make_async_copy(
            blks[s], rows_hbm.at[pl.ds(c * 4096, 4096)], sems_o[s]
        ).start()

    def wait_out(s):
        pltpu.make_async_copy(
            blks[s], rows_hbm.at[pl.ds(0, 4096)], sems_o[s]
        ).wait()

    start_in(cb, 0, GB)

    def tbody(t2, _):
        for ss in range(2):
            t = t2 * 2 + ss
            c0 = cb + t * GB
            wait_in(ss, GB)
            start_in(c0 + GB, 1 - ss, GB)
            for g in range(GB):
                @pl.when(t > 0)
                def _():
                    wait_out(g)

                _transpose_group(stages[ss], blks[g], 128, g * 128)
                start_out(c0 + g, g)
        return 0

    lax.fori_loop(0, GSTEPS // 2, tbody, 0)

    c0 = cb + (GSTEPS - 1) * GB
    wait_in(0, GB)
    for g in range(GB):
        wait_out(g)
        _transpose_group(stages[0], blks[g], 128, g * 128)
        start_out(c0 + g, g)
    for g in range(GB):
        wait_out(g)

    @pl.when(wid >= NW - GEXTRA)
    def _():
        c = GPW * NW + (wid - (NW - GEXTRA))
        start_in(c, 0, 1)
        wait_in(0, 1)
        _transpose_group(stages[0], blks[0], 128, 0)
        start_out(c, 0)
        wait_out(0)

    @pl.when(wid == NW - GEXTRA - 1)
    def _():
        pltpu.sync_copy(tail_hbm, tail_v)
        pltpu.sync_copy(
            tail_v, rows_hbm.at[pl.ds(GFULL * 4096, VTAIL * (D // 2))])


def _score_body(ctx_hbm, ids_hbm, rows2d_hbm, out_hbm,
                ids_v, ctx_v, rows_v, scores_v, out_v, sem_g, sems_o):
    wid = _worker_id()
    b0 = wid * BPW

    pltpu.sync_copy(ctx_hbm.at[pl.ds(b0, BPW), :], ctx_v)
    pltpu.sync_copy(ids_hbm.at[pl.ds(b0 * N, BPW * N)], ids_v)

    def fire_gathers(chunk, slot):
        for g in range(NGRP):
            pltpu.make_async_copy(
                rows2d_hbm.at[ids_v.at[pl.ds(chunk * ROWS + g * 128, 128)]],
                rows_v[slot].at[pl.ds(g * 128, 128), :],
                sem_g[slot],
            ).start()

    def wait_gathers(slot):
        for g in range(NGRP):
            pltpu.make_async_copy(
                rows2d_hbm.at[ids_v.at[pl.ds(g * 128, 128)]],
                rows_v[slot].at[pl.ds(g * 128, 128), :],
                sem_g[slot],
            ).wait()

    fire_gathers(0, 0)
    fire_gathers(1, 1)

    lane, perms = _perms()

    for chunk in range(NCHUNK):
        slot = chunk % 2
        wait_gathers(slot)

        def bbody(bb, _, _slot=slot, _chunk=chunk):
            bcol = lane + (_chunk * CB + bb * L)
            rbase = (lane + bb * L) * N

            for h in range(N // NHALF):
                rvecs = [rbase + (h * NHALF + i) for i in range(NHALF)]

                def dbody(j, accs, _rv=rvecs, _bcol=bcol):
                    q = ((lane + j) % L) + (j - (j % L))
                    de = q * 2
                    cve = plsc.load_gather(ctx_v, [_bcol, de])
                    cvo = plsc.load_gather(ctx_v, [_bcol, de + 1])
                    new = []
                    for i in range(NHALF):
                        w = plsc.load_gather(rows_v[_slot], [_rv[i], q])
                        a, b = plsc.unpack(
                            plsc.bitcast(w, jnp.bfloat16),
                            format=plsc.PackFormat.INTERLEAVED)
                        new.append(accs[i] + a * cve + b * cvo)
                    return tuple(new)

                accs = lax.fori_loop(
                    0, D // 2, dbody,
                    tuple(jnp.zeros((L,), jnp.float32) for _ in range(NHALF)))
                for i in range(NHALF):
                    scores_v[h * NHALF + i, :] = accs[i]

            m = scores_v[0, :]
            for n in range(1, N):
                m = jnp.maximum(m, scores_v[n, :])
            tot = jnp.zeros((L,), jnp.float32)
            for n in range(N):
                e = jnp.exp(scores_v[n, :] - m)
                scores_v[n, :] = e
                tot = tot + e
            r = 1.0 / tot
            blocal = lane + bb * L
            for n in range(N):
                ncol = jnp.full((L,), n, jnp.int32)
                plsc.store_scatter(out_v[_slot], [blocal, ncol],
                                   scores_v[n, :] * r)
            return 0

        lax.fori_loop(0, CB // L, bbody, 0)

        @pl.when(chunk >= 2)
        def _():
            pltpu.make_async_copy(
                out_v[slot], out_hbm.at[pl.ds(0, CB), :], sems_o[slot]
            ).wait()

        pltpu.make_async_copy(
            out_v[slot], out_hbm.at[pl.ds(b0 + chunk * CB, CB), :],
            sems_o[slot],
        ).start()

        if chunk + 2 < NCHUNK:
            fire_gathers(chunk + 2, slot)

    for slot in range(2):
        pltpu.make_async_copy(
            out_v[slot], out_hbm.at[pl.ds(0, CB), :], sems_o[slot]
        ).wait()


def _mesh():
    return plsc.VectorSubcoreMesh(
        core_axis_name="c", subcore_axis_name="s",
        num_cores=NC, num_subcores=NS)


@jax.jit
def _entity_posterior_sc(context_encoded, ids_flat, tab3, tail_flat):
    def relayout_wrap(tab3_hbm, tail_hbm, rows_hbm, s0, s1,
                      b0, b1, b2, b3, tv, si0, si1,
                      so0, so1, so2, so3):
        _relayout_body(tab3_hbm, tail_hbm, rows_hbm, (s0, s1),
                       (b0, b1, b2, b3), tv, (si0, si1),
                       (so0, so1, so2, so3))

    rows_lin = pl.kernel(
        relayout_wrap,
        out_type=jax.ShapeDtypeStruct((V * D // 2,), jnp.int32),
        mesh=_mesh(),
        scratch_types=(
            [pltpu.VMEM((D, GB * 128), jnp.float32) for _ in range(2)]
            + [pltpu.VMEM((4096,), jnp.int32) for _ in range(RING)]
            + [pltpu.VMEM((VTAIL * D // 2,), jnp.int32)]
            + [pltpu.SemaphoreType.DMA for _ in range(2)]
            + [pltpu.SemaphoreType.DMA for _ in range(RING)]
        ),
        compiler_params=pltpu.CompilerParams(needs_layout_passes=False),
        name="entity_table_relayout_sc",
    )(tab3, tail_flat)

    def score_wrap(ctx_hbm, ids_hbm, rows2d_hbm, out_hbm,
                   ids_v, ctx_v, r0, r1, o0, o1, sv, sg0, sg1, so0, so1):
        _score_body(ctx_hbm, ids_hbm, rows2d_hbm, out_hbm,
                    ids_v, ctx_v, (r0, r1), sv, (o0, o1),
                    (sg0, sg1), (so0, so1))

    return pl.kernel(
        score_wrap,
        out_type=jax.ShapeDtypeStruct((B, N), jnp.float32),
        mesh=_mesh(),
        scratch_types=(
            [pltpu.VMEM((BPW * N,), jnp.int32),
             pltpu.VMEM((BPW, D), jnp.float32)]
            + [pltpu.VMEM((ROWS, D // 2), jnp.int32) for _ in range(2)]
            + [pltpu.VMEM((CB, N), jnp.float32) for _ in range(2)]
            + [pltpu.VMEM((N, L), jnp.float32)]
            + [pltpu.SemaphoreType.DMA for _ in range(4)]
        ),
        compiler_params=pltpu.CompilerParams(
            needs_layout_passes=False, use_tc_tiling_on_sc=False),
        name="entity_posterior_sc",
    )(context_encoded, ids_flat, rows_lin.reshape(V, D // 2))


def kernel(context_encoded, entity_ids, entity_embeddings):
    tab3 = entity_embeddings.T.reshape(8, 8, V)
    tail_bf = entity_embeddings[GFULL * 128:, :].astype(jnp.bfloat16)
    tail_flat = jax.lax.bitcast_convert_type(
        tail_bf.reshape(-1, 2), jnp.int32).reshape(-1)
    ids_flat = entity_ids.reshape(-1)
    return _entity_posterior_sc(context_encoded, ids_flat, tab3, tail_flat)
